# baseline (speedup 1.0000x reference)
import functools

import jax
import jax.numpy as jnp
from jax import lax
from jax.experimental import pallas as pl
from jax.experimental.pallas import tpu as pltpu
from jax.experimental.pallas import tpu_sc as plsc

_PATCH_L = 16
_PATCH_C = 8
_NUM_PATCHES = 256
_OUT_DIM = 128
_PATCH_FLAT = _PATCH_L * _PATCH_C

_NC, _NS = 2, 16
_NW = _NC * _NS
_CHUNK = 8


def _sc_gather_patches(x2d, sl, sc):
    BL, C = x2d.shape
    NP = sl.shape[0]
    P = NP // _NW
    L = BL // _NW
    n_chunks = P // _CHUNK
    win = _PATCH_L + 8
    rows_per_chunk = _CHUNK * win

    mesh = plsc.VectorSubcoreMesh(core_axis_name="c", subcore_axis_name="s")

    @functools.partial(
        pl.kernel,
        out_type=jax.ShapeDtypeStruct((NP * _PATCH_FLAT,), jnp.float32),
        mesh=mesh,
        scratch_types=[
            pltpu.VMEM((P + 16,), jnp.int32),
            pltpu.VMEM((P + 16,), jnp.int32),
            pltpu.VMEM((rows_per_chunk, C), jnp.float32),
            pltpu.VMEM((P * _PATCH_FLAT,), jnp.float32),
            pltpu.SemaphoreType.DMA,
        ],
        compiler_params=pltpu.CompilerParams(needs_layout_passes=False,
                                             use_tc_tiling_on_sc=True),
    )
    def gather_kernel(x_hbm, sl_hbm, sc_hbm, out_hbm, slv, scv, rowsv,
                      outv, sem):
        wid = lax.axis_index("s") * _NC + lax.axis_index("c")
        base = wid * P
        pltpu.sync_copy(sl_hbm.at[pl.ds(base, P)], slv.at[pl.ds(0, P)])
        pltpu.sync_copy(sc_hbm.at[pl.ds(base, P)], scv.at[pl.ds(0, P)])
        iota = lax.iota(jnp.int32, 16)
        row_off = lax.shift_right_logical(iota, 3)
        col_off = lax.bitwise_and(iota, 7)
        xbase = wid * L

        def chunk_body(ci, carry):
            slc = slv[pl.ds(ci * _CHUNK, 16)]
            scc = scv[pl.ds(ci * _CHUNK, 16)]
            astart = jnp.minimum(lax.bitwise_and(slc, -8), L - win)
            roff = slc - astart
            for n in range(_CHUNK):
                pltpu.async_copy(
                    x_hbm.at[pl.ds(pl.multiple_of(xbase + astart[n], 8), win),
                             :],
                    rowsv.at[pl.ds(n * win, win), :], sem)
            for n in range(_CHUNK):
                q = ci * _CHUNK + n
                pltpu.make_async_copy(
                    x_hbm.at[pl.ds(0, win), :],
                    rowsv.at[pl.ds(n * win, win), :], sem).wait()
                cvec = jnp.full((16,), scc[n], jnp.int32) + col_off
                for k in range(_PATCH_FLAT // 16):
                    rvec = n * win + roff[n] + 2 * k + row_off
                    v = plsc.load_gather(rowsv, [rvec, cvec])
                    outv[pl.ds(q * _PATCH_FLAT + k * 16, 16)] = v
            return carry

        lax.fori_loop(0, n_chunks, chunk_body, 0)
        pltpu.sync_copy(outv, out_hbm.at[pl.ds(base * _PATCH_FLAT,
                                               P * _PATCH_FLAT)])

    return gather_kernel(x2d, sl, sc)


def _tc_mlp(pm, w1x, w1t, slf, inv_fs, b1, w2, b2):
    n = pm.shape[0]
    blk = 512
    grid = (n // blk,)

    def body(inv_ref, p_ref, w1x_ref, w1t_ref, sl_ref, b1_ref, w2_ref,
             b2_ref, o_ref):
        w1t = w1t_ref[...]
        ivec = lax.shift_right_logical(
            lax.broadcasted_iota(jnp.int32, (_PATCH_FLAT, 1), 0), 3
        ).astype(jnp.float32)
        s0 = jnp.sum(w1t, axis=0, keepdims=True)
        s1 = jnp.sum(w1t * ivec, axis=0, keepdims=True)
        inv = inv_ref[0, 0]
        h = jnp.dot(p_ref[...], w1x_ref[...],
                    preferred_element_type=jnp.float32)
        h = h + (sl_ref[...] * inv) * s0 + (inv * s1 + b1_ref[...])
        h = h * jax.nn.sigmoid(h)
        o_ref[...] = jnp.dot(h, w2_ref[...],
                             preferred_element_type=jnp.float32) + b2_ref[...]

    return pl.pallas_call(
        body,
        grid=grid,
        in_specs=[
            pl.BlockSpec(memory_space=pltpu.SMEM),
            pl.BlockSpec((blk, _PATCH_FLAT), lambda i: (i, 0)),
            pl.BlockSpec((_PATCH_FLAT, _OUT_DIM), lambda i: (0, 0)),
            pl.BlockSpec((_PATCH_FLAT, _OUT_DIM), lambda i: (0, 0)),
            pl.BlockSpec((blk, 1), lambda i: (i, 0)),
            pl.BlockSpec((1, _OUT_DIM), lambda i: (0, 0)),
            pl.BlockSpec((_OUT_DIM, _OUT_DIM), lambda i: (0, 0)),
            pl.BlockSpec((1, _OUT_DIM), lambda i: (0, 0)),
        ],
        out_specs=pl.BlockSpec((blk, _OUT_DIM), lambda i: (i, 0)),
        out_shape=jax.ShapeDtypeStruct((n, _OUT_DIM), jnp.float32),
    )(inv_fs, pm, w1x, w1t, slf, b1, w2, b2)


def kernel(x, fs, W1, b1, W2, b2):
    B, L, C = x.shape
    kidx = jax.random.key(42)
    kL, kC = jax.random.split(kidx)
    start_l = jax.random.randint(kL, (B, _NUM_PATCHES), 0, L - _PATCH_L + 1)
    start_c = jax.random.randint(kC, (B, _NUM_PATCHES), 0, C - _PATCH_C + 1)
    sl = start_l.reshape(-1).astype(jnp.int32)
    sc = start_c.reshape(-1).astype(jnp.int32)

    pm = _sc_gather_patches(x.reshape(B * L, C), sl, sc)
    pm = pm.reshape(B * _NUM_PATCHES, _PATCH_FLAT)

    w1r = W1.reshape(_PATCH_L, 2 * _PATCH_C, _OUT_DIM)
    w1x = w1r[:, :_PATCH_C, :].reshape(_PATCH_FLAT, _OUT_DIM)
    w1t = w1r[:, _PATCH_C:, :].reshape(_PATCH_FLAT, _OUT_DIM)
    inv_fs = (1.0 / jnp.asarray(fs).astype(jnp.float32)).reshape(1, 1)
    slf = sl.astype(jnp.float32).reshape(-1, 1)

    out = _tc_mlp(pm, w1x, w1t, slf, inv_fs, b1.reshape(1, -1), W2,
                  b2.reshape(1, -1))
    return out.reshape(B, _NUM_PATCHES, _OUT_DIM)


# speedup vs baseline: 1.4751x; 1.4751x over previous
import functools

import jax
import jax.numpy as jnp
from jax import lax
from jax.experimental import pallas as pl
from jax.experimental.pallas import tpu as pltpu
from jax.experimental.pallas import tpu_sc as plsc

_PATCH_L = 16
_PATCH_C = 8
_NUM_PATCHES = 256
_OUT_DIM = 128
_PATCH_FLAT = _PATCH_L * _PATCH_C

_NC, _NS = 2, 16
_NW = _NC * _NS
_CHUNK = 8


_CROWS = _CHUNK * 16


def _sc_gather_patches(xp, sl, sc, L, C):
    W = xp.shape[0]
    NP = sl.shape[0]
    P = NP // _NW
    n_chunks = P // _CHUNK
    lb_max = L // 128 - 1

    mesh = plsc.VectorSubcoreMesh(core_axis_name="c", subcore_axis_name="s")

    @functools.partial(
        pl.kernel,
        out_type=jax.ShapeDtypeStruct((NP * _PATCH_FLAT,), jnp.float32),
        mesh=mesh,
        scratch_types=[
            pltpu.VMEM((P + 16,), jnp.int32),
            pltpu.VMEM((P + 16,), jnp.int32),
            pltpu.VMEM((_CROWS,), jnp.int32),
            pltpu.VMEM((_CROWS,), jnp.int32),
            pltpu.VMEM((_CROWS, 128), jnp.float32),
            pltpu.VMEM((_CROWS, 128), jnp.float32),
            pltpu.VMEM((P * _PATCH_FLAT,), jnp.float32),
            pltpu.SemaphoreType.DMA,
            pltpu.SemaphoreType.DMA,
        ],
        compiler_params=pltpu.CompilerParams(needs_layout_passes=False),
    )
    def gather_kernel(x_hbm, sl_hbm, sc_hbm, out_hbm, slv, scv, idxa, idxb,
                      rowsa, rowsb, outv, sema, semb):
        wid = lax.axis_index("s") * _NC + lax.axis_index("c")
        base = wid * P
        pltpu.sync_copy(sl_hbm.at[pl.ds(base, P)], slv.at[pl.ds(0, P)])
        pltpu.sync_copy(sc_hbm.at[pl.ds(base, P)], scv.at[pl.ds(0, P)])
        rbase = wid * C
        iota = lax.iota(jnp.int32, 16)
        jh_j = lax.shift_right_logical(iota, 1)
        jh_h = lax.bitwise_and(iota, 1)
        jvec2 = lax.bitwise_and(iota, 7) * 2
        ivecs = [2 * k + lax.shift_right_logical(iota, 3)
                 for k in range(_PATCH_FLAT // 16)]

        def build_fire(ci, idxv, rowsv, sem):
            slc = slv[pl.ds(ci * _CHUNK, 16)]
            scc = scv[pl.ds(ci * _CHUNK, 16)]
            for n in range(_CHUNK):
                bcv = rbase + scc[n] + jh_j
                lbv = jnp.minimum(
                    lax.shift_right_logical(slc[n], 7) + jh_h, lb_max)
                rphys = (lax.shift_left(lax.shift_right_logical(bcv, 3), 9)
                         + lax.shift_left(lbv, 3) + lax.bitwise_and(bcv, 7))
                idxv[pl.ds(n * 16, 16)] = rphys
            pltpu.async_copy(x_hbm.at[idxv], rowsv, sem)

        def extract(ci, rowsv, sem):
            pltpu.make_async_copy(x_hbm.at[pl.ds(0, _CROWS)], rowsv,
                                  sem).wait()
            slc = slv[pl.ds(ci * _CHUNK, 16)]
            for n in range(_CHUNK):
                q = ci * _CHUNK + n
                t0 = jnp.full((16,), lax.bitwise_and(slc[n], 127), jnp.int32)
                rbn = jvec2 + n * 16
                for k in range(_PATCH_FLAT // 16):
                    t = t0 + ivecs[k]
                    rv = rbn + lax.shift_right_logical(t, 7)
                    cv = lax.bitwise_and(t, 127)
                    v = plsc.load_gather(rowsv, [rv, cv])
                    outv[pl.ds(q * _PATCH_FLAT + k * 16, 16)] = v

        build_fire(0, idxa, rowsa, sema)
        build_fire(1, idxb, rowsb, semb)

        def pair_body(i, carry):
            ci = i * 2
            extract(ci, rowsa, sema)

            @pl.when(ci + 2 < n_chunks)
            def _():
                build_fire(ci + 2, idxa, rowsa, sema)

            extract(ci + 1, rowsb, semb)

            @pl.when(ci + 3 < n_chunks)
            def _():
                build_fire(ci + 3, idxb, rowsb, semb)

            return carry

        lax.fori_loop(0, n_chunks // 2, pair_body, 0)
        pltpu.sync_copy(outv, out_hbm.at[pl.ds(base * _PATCH_FLAT,
                                               P * _PATCH_FLAT)])

    return gather_kernel(xp, sl, sc)


def _tc_mlp(pm, w1x, w1t, slf, inv_fs, b1, w2, b2):
    n = pm.shape[0]
    blk = 512
    grid = (n // blk,)

    def body(inv_ref, p_ref, w1x_ref, w1t_ref, sl_ref, b1_ref, w2_ref,
             b2_ref, o_ref):
        w1t = w1t_ref[...]
        ivec = lax.shift_right_logical(
            lax.broadcasted_iota(jnp.int32, (_PATCH_FLAT, 1), 0), 3
        ).astype(jnp.float32)
        s0 = jnp.sum(w1t, axis=0, keepdims=True)
        s1 = jnp.sum(w1t * ivec, axis=0, keepdims=True)
        inv = inv_ref[0, 0]
        h = jnp.dot(p_ref[...], w1x_ref[...],
                    preferred_element_type=jnp.float32)
        h = h + (sl_ref[...] * inv) * s0 + (inv * s1 + b1_ref[...])
        h = h * jax.nn.sigmoid(h)
        o_ref[...] = jnp.dot(h, w2_ref[...],
                             preferred_element_type=jnp.float32) + b2_ref[...]

    return pl.pallas_call(
        body,
        grid=grid,
        in_specs=[
            pl.BlockSpec(memory_space=pltpu.SMEM),
            pl.BlockSpec((blk, _PATCH_FLAT), lambda i: (i, 0)),
            pl.BlockSpec((_PATCH_FLAT, _OUT_DIM), lambda i: (0, 0)),
            pl.BlockSpec((_PATCH_FLAT, _OUT_DIM), lambda i: (0, 0)),
            pl.BlockSpec((blk, 1), lambda i: (i, 0)),
            pl.BlockSpec((1, _OUT_DIM), lambda i: (0, 0)),
            pl.BlockSpec((_OUT_DIM, _OUT_DIM), lambda i: (0, 0)),
            pl.BlockSpec((1, _OUT_DIM), lambda i: (0, 0)),
        ],
        out_specs=pl.BlockSpec((blk, _OUT_DIM), lambda i: (i, 0)),
        out_shape=jax.ShapeDtypeStruct((n, _OUT_DIM), jnp.float32),
    )(inv_fs, pm, w1x, w1t, slf, b1, w2, b2)


def kernel(x, fs, W1, b1, W2, b2):
    B, L, C = x.shape
    kidx = jax.random.key(42)
    kL, kC = jax.random.split(kidx)
    start_l = jax.random.randint(kL, (B, _NUM_PATCHES), 0, L - _PATCH_L + 1)
    start_c = jax.random.randint(kC, (B, _NUM_PATCHES), 0, C - _PATCH_C + 1)
    sl = start_l.reshape(-1).astype(jnp.int32)
    sc = start_c.reshape(-1).astype(jnp.int32)

    xp = (x.reshape(B, L // 128, 128, C // 8, 8).transpose(0, 3, 1, 4, 2)
          .reshape(B * L * C // 128, 128))
    pm = _sc_gather_patches(xp, sl, sc, L, C)
    pm = pm.reshape(B * _NUM_PATCHES, _PATCH_FLAT)

    w1r = W1.reshape(_PATCH_L, 2 * _PATCH_C, _OUT_DIM)
    w1x = w1r[:, :_PATCH_C, :].reshape(_PATCH_FLAT, _OUT_DIM)
    w1t = w1r[:, _PATCH_C:, :].reshape(_PATCH_FLAT, _OUT_DIM)
    inv_fs = (1.0 / jnp.asarray(fs).astype(jnp.float32)).reshape(1, 1)
    slf = sl.astype(jnp.float32).reshape(-1, 1)

    out = _tc_mlp(pm, w1x, w1t, slf, inv_fs, b1.reshape(1, -1), W2,
                  b2.reshape(1, -1))
    return out.reshape(B, _NUM_PATCHES, _OUT_DIM)


# speedup vs baseline: 1.9876x; 1.3475x over previous
import functools

import jax
import jax.numpy as jnp
import numpy as np
from jax import lax
from jax.experimental import pallas as pl
from jax.experimental.pallas import tpu as pltpu
from jax.experimental.pallas import tpu_sc as plsc

_PATCH_L = 16
_PATCH_C = 8
_NUM_PATCHES = 256
_OUT_DIM = 128
_PATCH_FLAT = _PATCH_L * _PATCH_C

_NC, _NS = 2, 16
_NW = _NC * _NS
_CHUNK = 8


_CROWS = _CHUNK * 16


def _sc_gather_patches(xp, sl, sc, L, C):
    W = xp.shape[0]
    NP = sl.shape[0]
    P = NP // _NW
    n_chunks = P // _CHUNK
    lb_max = L // 128 - 1

    mesh = plsc.VectorSubcoreMesh(core_axis_name="c", subcore_axis_name="s")

    @functools.partial(
        pl.kernel,
        out_type=jax.ShapeDtypeStruct((NP * _PATCH_FLAT,), jnp.float32),
        mesh=mesh,
        scratch_types=[
            pltpu.VMEM((P + 16,), jnp.int32),
            pltpu.VMEM((P + 16,), jnp.int32),
            pltpu.VMEM((_CROWS,), jnp.int32),
            pltpu.VMEM((_CROWS,), jnp.int32),
            pltpu.VMEM((_CROWS, 128), jnp.float32),
            pltpu.VMEM((_CROWS, 128), jnp.float32),
            pltpu.VMEM((P * _PATCH_FLAT,), jnp.float32),
            pltpu.SemaphoreType.DMA,
            pltpu.SemaphoreType.DMA,
        ],
        compiler_params=pltpu.CompilerParams(needs_layout_passes=False),
    )
    def gather_kernel(x_hbm, sl_hbm, sc_hbm, out_hbm, slv, scv, idxa, idxb,
                      rowsa, rowsb, outv, sema, semb):
        wid = lax.axis_index("s") * _NC + lax.axis_index("c")
        base = wid * P
        pltpu.sync_copy(sl_hbm.at[pl.ds(base, P)], slv.at[pl.ds(0, P)])
        pltpu.sync_copy(sc_hbm.at[pl.ds(base, P)], scv.at[pl.ds(0, P)])
        rbase = wid * C
        iota = lax.iota(jnp.int32, 16)
        jh_j = lax.shift_right_logical(iota, 1)
        jh_h = lax.bitwise_and(iota, 1)
        jvec2 = lax.bitwise_and(iota, 7) * 2
        ivecs = [2 * k + lax.shift_right_logical(iota, 3)
                 for k in range(_PATCH_FLAT // 16)]

        def build_fire(ci, idxv, rowsv, sem):
            slc = slv[pl.ds(ci * _CHUNK, 16)]
            scc = scv[pl.ds(ci * _CHUNK, 16)]
            for n in range(_CHUNK):
                bcv = rbase + scc[n] + jh_j
                lbv = jnp.minimum(
                    lax.shift_right_logical(slc[n], 7) + jh_h, lb_max)
                rphys = (lax.shift_left(lax.shift_right_logical(bcv, 3), 9)
                         + lax.shift_left(lbv, 3) + lax.bitwise_and(bcv, 7))
                idxv[pl.ds(n * 16, 16)] = rphys
            pltpu.async_copy(x_hbm.at[idxv], rowsv, sem)

        def extract(ci, rowsv, sem):
            pltpu.make_async_copy(x_hbm.at[pl.ds(0, _CROWS)], rowsv,
                                  sem).wait()
            slc = slv[pl.ds(ci * _CHUNK, 16)]
            for n in range(_CHUNK):
                q = ci * _CHUNK + n
                t0 = jnp.full((16,), lax.bitwise_and(slc[n], 127), jnp.int32)
                rbn = jvec2 + n * 16
                for k in range(_PATCH_FLAT // 16):
                    t = t0 + ivecs[k]
                    rv = rbn + lax.shift_right_logical(t, 7)
                    cv = lax.bitwise_and(t, 127)
                    v = plsc.load_gather(rowsv, [rv, cv])
                    outv[pl.ds(q * _PATCH_FLAT + k * 16, 16)] = v

        build_fire(0, idxa, rowsa, sema)
        build_fire(1, idxb, rowsb, semb)

        def pair_body(i, carry):
            ci = i * 2
            extract(ci, rowsa, sema)

            @pl.when(ci + 2 < n_chunks)
            def _():
                build_fire(ci + 2, idxa, rowsa, sema)

            extract(ci + 1, rowsb, semb)

            @pl.when(ci + 3 < n_chunks)
            def _():
                build_fire(ci + 3, idxb, rowsb, semb)

            return carry

        lax.fori_loop(0, n_chunks // 2, pair_body, 0)
        pltpu.sync_copy(outv, out_hbm.at[pl.ds(base * _PATCH_FLAT,
                                               P * _PATCH_FLAT)])

    return gather_kernel(xp, sl, sc)


def _tc_mlp(pm, w1x, w1t, slf, inv_fs, b1, w2, b2):
    n = pm.shape[0]
    blk = 512
    grid = (n // blk,)

    def body(inv_ref, p_ref, w1x_ref, w1t_ref, sl_ref, b1_ref, w2_ref,
             b2_ref, o_ref):
        w1t = w1t_ref[...]
        ivec = lax.shift_right_logical(
            lax.broadcasted_iota(jnp.int32, (_PATCH_FLAT, 1), 0), 3
        ).astype(jnp.float32)
        s0 = jnp.sum(w1t, axis=0, keepdims=True)
        s1 = jnp.sum(w1t * ivec, axis=0, keepdims=True)
        inv = inv_ref[0, 0]
        h = jnp.dot(p_ref[...], w1x_ref[...],
                    preferred_element_type=jnp.float32)
        h = h + (sl_ref[...] * inv) * s0 + (inv * s1 + b1_ref[...])
        h = h * jax.nn.sigmoid(h)
        o_ref[...] = jnp.dot(h, w2_ref[...],
                             preferred_element_type=jnp.float32) + b2_ref[...]

    return pl.pallas_call(
        body,
        grid=grid,
        in_specs=[
            pl.BlockSpec(memory_space=pltpu.SMEM),
            pl.BlockSpec((blk, _PATCH_FLAT), lambda i: (i, 0)),
            pl.BlockSpec((_PATCH_FLAT, _OUT_DIM), lambda i: (0, 0)),
            pl.BlockSpec((_PATCH_FLAT, _OUT_DIM), lambda i: (0, 0)),
            pl.BlockSpec((blk, 1), lambda i: (i, 0)),
            pl.BlockSpec((1, _OUT_DIM), lambda i: (0, 0)),
            pl.BlockSpec((_OUT_DIM, _OUT_DIM), lambda i: (0, 0)),
            pl.BlockSpec((1, _OUT_DIM), lambda i: (0, 0)),
        ],
        out_specs=pl.BlockSpec((blk, _OUT_DIM), lambda i: (i, 0)),
        out_shape=jax.ShapeDtypeStruct((n, _OUT_DIM), jnp.float32),
    )(inv_fs, pm, w1x, w1t, slf, b1, w2, b2)


@functools.lru_cache(maxsize=None)
def _patch_starts(B, L, C):
    with jax.ensure_compile_time_eval():
        kidx = jax.random.key(42)
        kL, kC = jax.random.split(kidx)
        start_l = jax.random.randint(kL, (B, _NUM_PATCHES), 0,
                                     L - _PATCH_L + 1)
        start_c = jax.random.randint(kC, (B, _NUM_PATCHES), 0,
                                     C - _PATCH_C + 1)
        sl = np.asarray(start_l, np.int32).reshape(-1)
        sc = np.asarray(start_c, np.int32).reshape(-1)
    return sl, sc


def kernel(x, fs, W1, b1, W2, b2):
    B, L, C = x.shape
    sl_np, sc_np = _patch_starts(B, L, C)
    sl = jnp.asarray(sl_np)
    sc = jnp.asarray(sc_np)

    xp = (x.reshape(B, L // 128, 128, C // 8, 8).transpose(0, 3, 1, 4, 2)
          .reshape(B * L * C // 128, 128))
    pm = _sc_gather_patches(xp, sl, sc, L, C)
    pm = pm.reshape(B * _NUM_PATCHES, _PATCH_FLAT)

    w1r = W1.reshape(_PATCH_L, 2 * _PATCH_C, _OUT_DIM)
    w1x = w1r[:, :_PATCH_C, :].reshape(_PATCH_FLAT, _OUT_DIM)
    w1t = w1r[:, _PATCH_C:, :].reshape(_PATCH_FLAT, _OUT_DIM)
    inv_fs = (1.0 / jnp.asarray(fs).astype(jnp.float32)).reshape(1, 1)
    slf = jnp.asarray(sl_np.astype(np.float32).reshape(-1, 1))

    out = _tc_mlp(pm, w1x, w1t, slf, inv_fs, b1.reshape(1, -1), W2,
                  b2.reshape(1, -1))
    return out.reshape(B, _NUM_PATCHES, _OUT_DIM)


# speedup vs baseline: 2.2254x; 1.1196x over previous
import functools

import jax
import jax.numpy as jnp
import numpy as np
from jax import lax
from jax.experimental import pallas as pl
from jax.experimental.pallas import tpu as pltpu
from jax.experimental.pallas import tpu_sc as plsc

_PATCH_L = 16
_PATCH_C = 8
_NUM_PATCHES = 256
_OUT_DIM = 128
_PATCH_FLAT = _PATCH_L * _PATCH_C

_NC, _NS = 2, 16
_NW = _NC * _NS
_CHUNK = 8


_CROWS = _CHUNK * 16
_NBUF = 4


def _sc_gather_patches(xp, sl, sc, L, C):
    W = xp.shape[0]
    NP = sl.shape[0]
    P = NP // _NW
    n_chunks = P // _CHUNK
    lb_max = L // 128 - 1

    mesh = plsc.VectorSubcoreMesh(core_axis_name="c", subcore_axis_name="s")

    @functools.partial(
        pl.kernel,
        out_type=jax.ShapeDtypeStruct((NP * _PATCH_FLAT,), jnp.float32),
        mesh=mesh,
        scratch_types=[
            pltpu.VMEM((P + 16,), jnp.int32),
            pltpu.VMEM((P + 16,), jnp.int32),
            pltpu.VMEM((n_chunks, _CROWS), jnp.int32),
            pltpu.VMEM((_NBUF, _CROWS, 128), jnp.float32),
            pltpu.VMEM((P * _PATCH_FLAT,), jnp.float32),
            [pltpu.SemaphoreType.DMA] * _NBUF,
        ],
        compiler_params=pltpu.CompilerParams(needs_layout_passes=False),
    )
    def gather_kernel(x_hbm, sl_hbm, sc_hbm, out_hbm, slv, scv, idxall,
                      rows, outv, sems):
        wid = lax.axis_index("s") * _NC + lax.axis_index("c")
        base = wid * P
        pltpu.sync_copy(sl_hbm.at[pl.ds(base, P)], slv.at[pl.ds(0, P)])
        pltpu.sync_copy(sc_hbm.at[pl.ds(base, P)], scv.at[pl.ds(0, P)])
        rbase = wid * C
        iota = lax.iota(jnp.int32, 16)
        jh_j = lax.shift_right_logical(iota, 1)
        jh_h = lax.bitwise_and(iota, 1)
        jvec2 = lax.bitwise_and(iota, 7) * 2
        ivecs = [2 * k + lax.shift_right_logical(iota, 3)
                 for k in range(_PATCH_FLAT // 16)]

        def build_idx(ci, carry):
            slc = slv[pl.ds(ci * _CHUNK, 16)]
            scc = scv[pl.ds(ci * _CHUNK, 16)]
            for n in range(_CHUNK):
                bcv = rbase + scc[n] + jh_j
                lbv = jnp.minimum(
                    lax.shift_right_logical(slc[n], 7) + jh_h, lb_max)
                rphys = (lax.shift_left(lax.shift_right_logical(bcv, 3), 9)
                         + lax.shift_left(lbv, 3) + lax.bitwise_and(bcv, 7))
                idxall[ci, pl.ds(n * 16, 16)] = rphys
            return carry

        lax.fori_loop(0, n_chunks, build_idx, 0)

        def fire(ci, b):
            pltpu.async_copy(x_hbm.at[idxall.at[ci]], rows.at[b], sems[b])

        def extract(ci, b):
            pltpu.make_async_copy(x_hbm.at[pl.ds(0, _CROWS)], rows.at[b],
                                  sems[b]).wait()
            slc = slv[pl.ds(ci * _CHUNK, 16)]
            for n in range(_CHUNK):
                q = ci * _CHUNK + n
                t0 = jnp.full((16,), lax.bitwise_and(slc[n], 127), jnp.int32)
                rbn = jvec2 + n * 16
                for k in range(_PATCH_FLAT // 16):
                    t = t0 + ivecs[k]
                    rv = rbn + lax.shift_right_logical(t, 7)
                    cv = lax.bitwise_and(t, 127)
                    v = plsc.load_gather(rows.at[b], [rv, cv])
                    outv[pl.ds(q * _PATCH_FLAT + k * 16, 16)] = v

        for b in range(_NBUF):
            fire(b, b)

        def ring_body(i, carry):
            ci = i * _NBUF
            for b in range(_NBUF):
                extract(ci + b, b)

                @pl.when(ci + b + _NBUF < n_chunks)
                def _():
                    fire(ci + b + _NBUF, b)

            return carry

        lax.fori_loop(0, n_chunks // _NBUF, ring_body, 0)
        pltpu.sync_copy(outv, out_hbm.at[pl.ds(base * _PATCH_FLAT,
                                               P * _PATCH_FLAT)])

    return gather_kernel(xp, sl, sc)


def _tc_mlp(pm, w1x, w1t, slf, inv_fs, b1, w2, b2):
    n = pm.shape[0]
    blk = 512
    grid = (n // blk,)

    def body(inv_ref, p_ref, w1x_ref, w1t_ref, sl_ref, b1_ref, w2_ref,
             b2_ref, o_ref):
        w1t = w1t_ref[...]
        ivec = lax.shift_right_logical(
            lax.broadcasted_iota(jnp.int32, (_PATCH_FLAT, 1), 0), 3
        ).astype(jnp.float32)
        s0 = jnp.sum(w1t, axis=0, keepdims=True)
        s1 = jnp.sum(w1t * ivec, axis=0, keepdims=True)
        inv = inv_ref[0, 0]
        h = jnp.dot(p_ref[...], w1x_ref[...],
                    preferred_element_type=jnp.float32)
        h = h + (sl_ref[...] * inv) * s0 + (inv * s1 + b1_ref[...])
        h = h * jax.nn.sigmoid(h)
        o_ref[...] = jnp.dot(h, w2_ref[...],
                             preferred_element_type=jnp.float32) + b2_ref[...]

    return pl.pallas_call(
        body,
        grid=grid,
        in_specs=[
            pl.BlockSpec(memory_space=pltpu.SMEM),
            pl.BlockSpec((blk, _PATCH_FLAT), lambda i: (i, 0)),
            pl.BlockSpec((_PATCH_FLAT, _OUT_DIM), lambda i: (0, 0)),
            pl.BlockSpec((_PATCH_FLAT, _OUT_DIM), lambda i: (0, 0)),
            pl.BlockSpec((blk, 1), lambda i: (i, 0)),
            pl.BlockSpec((1, _OUT_DIM), lambda i: (0, 0)),
            pl.BlockSpec((_OUT_DIM, _OUT_DIM), lambda i: (0, 0)),
            pl.BlockSpec((1, _OUT_DIM), lambda i: (0, 0)),
        ],
        out_specs=pl.BlockSpec((blk, _OUT_DIM), lambda i: (i, 0)),
        out_shape=jax.ShapeDtypeStruct((n, _OUT_DIM), jnp.float32),
    )(inv_fs, pm, w1x, w1t, slf, b1, w2, b2)


@functools.lru_cache(maxsize=None)
def _patch_starts(B, L, C):
    try:
        with jax.ensure_compile_time_eval():
            kidx = jax.random.key(42)
            kL, kC = jax.random.split(kidx)
            start_l = jax.random.randint(kL, (B, _NUM_PATCHES), 0,
                                         L - _PATCH_L + 1)
            start_c = jax.random.randint(kC, (B, _NUM_PATCHES), 0,
                                         C - _PATCH_C + 1)
            sl = np.asarray(start_l, np.int32).reshape(-1)
            sc = np.asarray(start_c, np.int32).reshape(-1)
        return sl, sc
    except Exception:
        return None


def _patch_starts_traced(B, L, C):
    kidx = jax.random.key(42)
    kL, kC = jax.random.split(kidx)
    start_l = jax.random.randint(kL, (B, _NUM_PATCHES), 0, L - _PATCH_L + 1)
    start_c = jax.random.randint(kC, (B, _NUM_PATCHES), 0, C - _PATCH_C + 1)
    return (start_l.reshape(-1).astype(jnp.int32),
            start_c.reshape(-1).astype(jnp.int32))


def kernel(x, fs, W1, b1, W2, b2):
    B, L, C = x.shape
    starts = _patch_starts(B, L, C)
    if starts is None:
        sl, sc = _patch_starts_traced(B, L, C)
        slf = sl.astype(jnp.float32).reshape(-1, 1)
    else:
        sl_np, sc_np = starts
        sl = jnp.asarray(sl_np)
        sc = jnp.asarray(sc_np)
        slf = jnp.asarray(sl_np.astype(np.float32).reshape(-1, 1))

    xp = (x.reshape(B, L // 128, 128, C // 8, 8).transpose(0, 3, 1, 4, 2)
          .reshape(B * L * C // 128, 128))
    pm = _sc_gather_patches(xp, sl, sc, L, C)
    pm = pm.reshape(B * _NUM_PATCHES, _PATCH_FLAT)

    w1r = W1.reshape(_PATCH_L, 2 * _PATCH_C, _OUT_DIM)
    w1x = w1r[:, :_PATCH_C, :].reshape(_PATCH_FLAT, _OUT_DIM)
    w1t = w1r[:, _PATCH_C:, :].reshape(_PATCH_FLAT, _OUT_DIM)
    inv_fs = (1.0 / jnp.asarray(fs).astype(jnp.float32)).reshape(1, 1)

    out = _tc_mlp(pm, w1x, w1t, slf, inv_fs, b1.reshape(1, -1), W2,
                  b2.reshape(1, -1))
    return out.reshape(B, _NUM_PATCHES, _OUT_DIM)


# speedup vs baseline: 2.4881x; 1.1181x over previous
import functools

import jax
import jax.numpy as jnp
import numpy as np
from jax import lax
from jax.experimental import pallas as pl
from jax.experimental.pallas import tpu as pltpu
from jax.experimental.pallas import tpu_sc as plsc

_PATCH_L = 16
_PATCH_C = 8
_NUM_PATCHES = 256
_OUT_DIM = 128
_PATCH_FLAT = _PATCH_L * _PATCH_C

_NC, _NS = 2, 16
_NW = _NC * _NS
_CHUNK = 8


_CROWS = _CHUNK * 16
_NBUF = 4


def _sc_gather_patches(xp, sl, sc, L, C):
    W = xp.shape[0]
    NP = sl.shape[0]
    P = NP // _NW
    n_chunks = P // _CHUNK
    lb_max = L // 128 - 1

    mesh = plsc.VectorSubcoreMesh(core_axis_name="c", subcore_axis_name="s")

    @functools.partial(
        pl.kernel,
        out_type=jax.ShapeDtypeStruct((NP * _PATCH_FLAT,), jnp.float32),
        mesh=mesh,
        scratch_types=[
            pltpu.VMEM((P + 16,), jnp.int32),
            pltpu.VMEM((P + 16,), jnp.int32),
            pltpu.VMEM((n_chunks, _CROWS), jnp.int32),
            pltpu.VMEM((_NBUF, _CROWS, 128), jnp.float32),
            pltpu.VMEM((P * _PATCH_FLAT,), jnp.float32),
            [pltpu.SemaphoreType.DMA] * _NBUF,
        ],
        compiler_params=pltpu.CompilerParams(needs_layout_passes=False),
    )
    def gather_kernel(x_hbm, sl_hbm, sc_hbm, out_hbm, slv, scv, idxall,
                      rows, outv, sems):
        wid = lax.axis_index("s") * _NC + lax.axis_index("c")
        base = wid * P
        pltpu.sync_copy(sl_hbm.at[pl.ds(base, P)], slv.at[pl.ds(0, P)])
        pltpu.sync_copy(sc_hbm.at[pl.ds(base, P)], scv.at[pl.ds(0, P)])
        rbase = wid * C
        iota = lax.iota(jnp.int32, 16)
        jh_j = lax.shift_right_logical(iota, 1)
        jh_h = lax.bitwise_and(iota, 1)
        jvec2 = lax.bitwise_and(iota, 7) * 2
        ivecs = [2 * k + lax.shift_right_logical(iota, 3)
                 for k in range(_PATCH_FLAT // 16)]

        def build_idx(ci, carry):
            slc = slv[pl.ds(ci * _CHUNK, 16)]
            scc = scv[pl.ds(ci * _CHUNK, 16)]
            for n in range(_CHUNK):
                bcv = rbase + scc[n] + jh_j
                lbv = jnp.minimum(
                    lax.shift_right_logical(slc[n], 7) + jh_h, lb_max)
                rphys = (lax.shift_left(lax.shift_right_logical(bcv, 3), 9)
                         + lax.shift_left(lbv, 3) + lax.bitwise_and(bcv, 7))
                idxall[ci, pl.ds(n * 16, 16)] = rphys
            return carry

        lax.fori_loop(0, n_chunks, build_idx, 0)

        def fire(ci, b):
            pltpu.async_copy(x_hbm.at[idxall.at[ci]], rows.at[b], sems[b])

        def extract(ci, b):
            pltpu.make_async_copy(x_hbm.at[pl.ds(0, _CROWS)], rows.at[b],
                                  sems[b]).wait()
            slc = slv[pl.ds(ci * _CHUNK, 16)]
            for n in range(_CHUNK):
                q = ci * _CHUNK + n
                t0 = jnp.full((16,), lax.bitwise_and(slc[n], 127), jnp.int32)
                rbn = jvec2 + n * 16
                for k in range(_PATCH_FLAT // 16):
                    t = t0 + ivecs[k]
                    rv = rbn + lax.shift_right_logical(t, 7)
                    cv = lax.bitwise_and(t, 127)
                    v = plsc.load_gather(rows.at[b], [rv, cv])
                    outv[pl.ds(q * _PATCH_FLAT + k * 16, 16)] = v

        for b in range(_NBUF):
            fire(b, b)

        def ring_body(i, carry):
            ci = i * _NBUF
            for b in range(_NBUF):
                extract(ci + b, b)

                @pl.when(ci + b + _NBUF < n_chunks)
                def _():
                    fire(ci + b + _NBUF, b)

            return carry

        lax.fori_loop(0, n_chunks // _NBUF, ring_body, 0)
        pltpu.sync_copy(outv, out_hbm.at[pl.ds(base * _PATCH_FLAT,
                                               P * _PATCH_FLAT)])

    return gather_kernel(xp, sl, sc)


def _tc_mlp(pm, w1x, w1t, slf, inv_fs, b1, w2, b2):
    n = pm.shape[0]
    blk = 2048
    grid = (n // blk,)

    def body(inv_ref, p_ref, w1x_ref, w1t_ref, sl_ref, b1_ref, w2_ref,
             b2_ref, o_ref):
        w1t = w1t_ref[...]
        ivec = lax.shift_right_logical(
            lax.broadcasted_iota(jnp.int32, (_PATCH_FLAT, 1), 0), 3
        ).astype(jnp.float32)
        s0 = jnp.sum(w1t, axis=0, keepdims=True)
        s1 = jnp.sum(w1t * ivec, axis=0, keepdims=True)
        inv = inv_ref[0, 0]
        h = jnp.dot(p_ref[...], w1x_ref[...],
                    preferred_element_type=jnp.float32)
        h = h + (sl_ref[...] * inv) * s0 + (inv * s1 + b1_ref[...])
        h = h * jax.nn.sigmoid(h)
        o_ref[...] = jnp.dot(h, w2_ref[...],
                             preferred_element_type=jnp.float32) + b2_ref[...]

    return pl.pallas_call(
        body,
        grid=grid,
        in_specs=[
            pl.BlockSpec(memory_space=pltpu.SMEM),
            pl.BlockSpec((blk, _PATCH_FLAT), lambda i: (i, 0)),
            pl.BlockSpec((_PATCH_FLAT, _OUT_DIM), lambda i: (0, 0)),
            pl.BlockSpec((_PATCH_FLAT, _OUT_DIM), lambda i: (0, 0)),
            pl.BlockSpec((blk, 1), lambda i: (i, 0)),
            pl.BlockSpec((1, _OUT_DIM), lambda i: (0, 0)),
            pl.BlockSpec((_OUT_DIM, _OUT_DIM), lambda i: (0, 0)),
            pl.BlockSpec((1, _OUT_DIM), lambda i: (0, 0)),
        ],
        out_specs=pl.BlockSpec((blk, _OUT_DIM), lambda i: (i, 0)),
        out_shape=jax.ShapeDtypeStruct((n, _OUT_DIM), jnp.float32),
    )(inv_fs, pm, w1x, w1t, slf, b1, w2, b2)


@functools.lru_cache(maxsize=None)
def _patch_starts(B, L, C):
    try:
        with jax.ensure_compile_time_eval():
            kidx = jax.random.key(42)
            kL, kC = jax.random.split(kidx)
            start_l = jax.random.randint(kL, (B, _NUM_PATCHES), 0,
                                         L - _PATCH_L + 1)
            start_c = jax.random.randint(kC, (B, _NUM_PATCHES), 0,
                                         C - _PATCH_C + 1)
            sl = np.asarray(start_l, np.int32).reshape(-1)
            sc = np.asarray(start_c, np.int32).reshape(-1)
        return sl, sc
    except Exception:
        return None


def _patch_starts_traced(B, L, C):
    kidx = jax.random.key(42)
    kL, kC = jax.random.split(kidx)
    start_l = jax.random.randint(kL, (B, _NUM_PATCHES), 0, L - _PATCH_L + 1)
    start_c = jax.random.randint(kC, (B, _NUM_PATCHES), 0, C - _PATCH_C + 1)
    return (start_l.reshape(-1).astype(jnp.int32),
            start_c.reshape(-1).astype(jnp.int32))


def kernel(x, fs, W1, b1, W2, b2):
    B, L, C = x.shape
    starts = _patch_starts(B, L, C)
    if starts is None:
        sl, sc = _patch_starts_traced(B, L, C)
        slf = sl.astype(jnp.float32).reshape(-1, 1)
    else:
        sl_np, sc_np = starts
        sl = jnp.asarray(sl_np)
        sc = jnp.asarray(sc_np)
        slf = jnp.asarray(sl_np.astype(np.float32).reshape(-1, 1))

    xp = (x.reshape(B, L // 128, 128, C // 8, 8).transpose(0, 3, 1, 4, 2)
          .reshape(B * L * C // 128, 128))
    pm = _sc_gather_patches(xp, sl, sc, L, C)
    pm = pm.reshape(B * _NUM_PATCHES, _PATCH_FLAT)

    w1r = W1.reshape(_PATCH_L, 2 * _PATCH_C, _OUT_DIM)
    w1x = w1r[:, :_PATCH_C, :].reshape(_PATCH_FLAT, _OUT_DIM)
    w1t = w1r[:, _PATCH_C:, :].reshape(_PATCH_FLAT, _OUT_DIM)
    inv_fs = (1.0 / jnp.asarray(fs).astype(jnp.float32)).reshape(1, 1)

    out = _tc_mlp(pm, w1x, w1t, slf, inv_fs, b1.reshape(1, -1), W2,
                  b2.reshape(1, -1))
    return out.reshape(B, _NUM_PATCHES, _OUT_DIM)
